# baseline (speedup 1.0000x reference)
import functools

import jax
import jax.numpy as jnp
from jax import lax
from jax.experimental import pallas as pl
from jax.experimental.pallas import tpu as pltpu
from jax.experimental.pallas import tpu_sc as plsc

N, E, D, R = 10000, 320000, 128, 8
NR = N * R
EPS = 1e-5

NC, NS = 2, 16
NW = NC * NS
CB = 80
BC = 2000
EPW = E // NW
EPC = E // NS
NRT = NR // NS
NRT_PAD = 5008
ART = 624
BN = 1000
NB = N // BN


def _iota16():
    return lax.broadcasted_iota(jnp.int32, (16,), 0)


def _sc_body(src_hbm, dst_hbm, typ_hbm, yflat_hbm, acc_out,
             cnt_sh, acc_sh, rowbuf, rowbuf1,
             srcB, dstB, typB,
             gidxb, dstb, widxb, gidxb1, dstb1, widxb1,
             wbuf, onesb, cidxb, cbuf, sem, sem1, sem2, sem3):
    c = lax.axis_index("c")
    s = lax.axis_index("s")
    wid = c * NS + s
    i16 = _iota16()
    z16 = jnp.zeros((16,), jnp.float32)
    o16 = jnp.ones((16,), jnp.float32)
    zi16 = jnp.zeros((16,), jnp.int32)

    def zero_row(e, _):
        for k in range(8):
            rowbuf[e, pl.ds(k * 16, 16)] = z16
        return 0
    lax.fori_loop(0, CB, zero_row, 0)

    def zero_cbuf(i, _):
        cbuf[pl.ds(i * 16, 16)] = z16
        return 0
    lax.fori_loop(0, NRT_PAD // 16, zero_cbuf, 0)

    for j in range(CB // 16):
        onesb[pl.ds(j * 16, 16)] = o16

    pltpu.sync_copy(cbuf.at[pl.ds(0, NRT)], cnt_sh.at[pl.ds(s * NRT, NRT)])
    for j in range(ART // CB):
        pltpu.sync_copy(rowbuf, acc_sh.at[pl.ds(s * ART + j * CB, CB)])
    rem = ART - (ART // CB) * CB
    if rem:
        pltpu.sync_copy(rowbuf.at[pl.ds(0, rem)],
                        acc_sh.at[pl.ds(s * ART + (ART // CB) * CB, rem)])

    @pl.when(s == NS - 1)
    def _zero_tail():
        pltpu.sync_copy(rowbuf.at[pl.ds(0, N - NS * ART)],
                        acc_sh.at[pl.ds(NS * ART, N - NS * ART)])
    plsc.subcore_barrier()

    cbase = s * EPC
    for b in range(EPC // BC):
        pltpu.sync_copy(dst_hbm.at[pl.ds(cbase + b * BC, BC)], dstB)
        pltpu.sync_copy(typ_hbm.at[pl.ds(cbase + b * BC, BC)], typB)

        def count_chunk(i, _):
            for j in range(CB // 16):
                off = i * CB + j * 16
                d16 = dstB[pl.ds(off, 16)]
                t16 = typB[pl.ds(off, 16)]
                cidxb[pl.ds(j * 16, 16)] = d16 * R + t16
            pltpu.sync_copy(onesb, cnt_sh.at[cidxb], add=True)
            return 0
        lax.fori_loop(0, BC // CB, count_chunk, 0)
    plsc.subcore_barrier()

    pltpu.sync_copy(cnt_sh.at[pl.ds(s * NRT, NRT)], cbuf.at[pl.ds(0, NRT)])

    def inv_chunk(i, _):
        cv = cbuf[pl.ds(i * 16, 16)]
        cbuf[pl.ds(i * 16, 16)] = 1.0 / jnp.maximum(cv, 1.0)
        return 0
    lax.fori_loop(0, NRT_PAD // 16, inv_chunk, 0)
    pltpu.sync_copy(cbuf.at[pl.ds(0, NRT)], cnt_sh.at[pl.ds(s * NRT, NRT)])
    plsc.subcore_barrier()

    base = wid * EPW

    def idx_chunk(t, gb, db, wb):
        def g(j, _):
            off = t * CB + j * 16
            s16 = srcB[pl.ds(off, 16)]
            d16 = dstB[pl.ds(off, 16)]
            t16 = typB[pl.ds(off, 16)]
            gb[pl.ds(j * 16, 16)] = t16 * N + s16
            db[pl.ds(j * 16, 16)] = d16
            wb[pl.ds(j * 16, 16)] = d16 * R + t16
            return 0
        lax.fori_loop(0, CB // 16, g, 0)

    def scale(rb, wi):
        pltpu.sync_copy(cnt_sh.at[wi], wbuf)

        def scale_group(j, _):
            wv = wbuf[pl.ds(j * 16, 16)]
            for e in range(16):
                w = wv[e]
                row = j * 16 + e
                for k in range(8):
                    rb[row, pl.ds(k * 16, 16)] = rb[row, pl.ds(k * 16, 16)] * w
            return 0
        lax.fori_loop(0, CB // 16, scale_group, 0)

    def g_wait(rb, gb):
        pltpu.make_async_copy(yflat_hbm.at[gb], rb, sem).wait()

    def g_wait1(rb, gb):
        pltpu.make_async_copy(yflat_hbm.at[gb], rb, sem1).wait()

    CPB = BC // CB

    def block_body(b, _):
        pltpu.sync_copy(src_hbm.at[pl.ds(base + b * BC, BC)], srcB)
        pltpu.sync_copy(dst_hbm.at[pl.ds(base + b * BC, BC)], dstB)
        pltpu.sync_copy(typ_hbm.at[pl.ds(base + b * BC, BC)], typB)

        idx_chunk(0, gidxb, dstb, widxb)
        pltpu.async_copy(yflat_hbm.at[gidxb], rowbuf, sem)
        idx_chunk(1, gidxb1, dstb1, widxb1)
        pltpu.async_copy(yflat_hbm.at[gidxb1], rowbuf1, sem1)

        def pipe_body(i, _):
            t = 2 * i
            g_wait(rowbuf, gidxb)
            scale(rowbuf, widxb)
            d0 = pltpu.async_copy(rowbuf, acc_sh.at[dstb], sem2, add=True)
            g_wait1(rowbuf1, gidxb1)
            scale(rowbuf1, widxb1)
            d1 = pltpu.async_copy(rowbuf1, acc_sh.at[dstb1], sem3, add=True)
            d0.wait()
            idx_chunk(t + 2, gidxb, dstb, widxb)
            pltpu.async_copy(yflat_hbm.at[gidxb], rowbuf, sem)
            d1.wait()
            idx_chunk(t + 3, gidxb1, dstb1, widxb1)
            pltpu.async_copy(yflat_hbm.at[gidxb1], rowbuf1, sem1)
            return 0
        lax.fori_loop(0, (CPB - 3) // 2, pipe_body, 0)

        g_wait(rowbuf, gidxb)
        scale(rowbuf, widxb)
        d0 = pltpu.async_copy(rowbuf, acc_sh.at[dstb], sem2, add=True)
        g_wait1(rowbuf1, gidxb1)
        scale(rowbuf1, widxb1)
        pltpu.sync_copy(rowbuf1, acc_sh.at[dstb1], add=True)
        d0.wait()
        idx_chunk(CPB - 1, gidxb, dstb, widxb)
        pltpu.async_copy(yflat_hbm.at[gidxb], rowbuf, sem)
        g_wait(rowbuf, gidxb)
        scale(rowbuf, widxb)
        pltpu.sync_copy(rowbuf, acc_sh.at[dstb], add=True)
        return 0
    lax.fori_loop(0, EPW // BC, block_body, 0)
    plsc.subcore_barrier()

    pltpu.sync_copy(acc_sh.at[pl.ds(s * ART, ART)],
                    acc_out.at[c, pl.ds(s * ART, ART)])

    @pl.when(s == NS - 1)
    def _export_tail():
        pltpu.sync_copy(acc_sh.at[pl.ds(NS * ART, N - NS * ART)],
                        acc_out.at[c, pl.ds(NS * ART, N - NS * ART)])


_sc_kernel = functools.partial(
    pl.kernel,
    out_type=jax.ShapeDtypeStruct((NC, N, D), jnp.float32),
    mesh=plsc.VectorSubcoreMesh(core_axis_name="c", subcore_axis_name="s",
                                num_cores=NC, num_subcores=NS),
    scratch_types=[
        pltpu.VMEM_SHARED((NR,), jnp.float32),
        pltpu.VMEM_SHARED((N, D), jnp.float32),
        pltpu.VMEM((CB, D), jnp.float32),
        pltpu.VMEM((CB, D), jnp.float32),
        pltpu.VMEM((BC,), jnp.int32),
        pltpu.VMEM((BC,), jnp.int32),
        pltpu.VMEM((BC,), jnp.int32),
        pltpu.VMEM((CB,), jnp.int32),
        pltpu.VMEM((CB,), jnp.int32),
        pltpu.VMEM((CB,), jnp.int32),
        pltpu.VMEM((CB,), jnp.int32),
        pltpu.VMEM((CB,), jnp.int32),
        pltpu.VMEM((CB,), jnp.int32),
        pltpu.VMEM((CB,), jnp.float32),
        pltpu.VMEM((CB,), jnp.float32),
        pltpu.VMEM((CB,), jnp.int32),
        pltpu.VMEM((NRT_PAD,), jnp.float32),
        pltpu.SemaphoreType.DMA,
        pltpu.SemaphoreType.DMA,
        pltpu.SemaphoreType.DMA,
        pltpu.SemaphoreType.DMA,
    ],
)(_sc_body)


def _mm_body(x_ref, wc_ref, ry_ref, yr_ref):
    xb = x_ref[...]
    ry_ref[...] = jnp.dot(xb, wc_ref[0], preferred_element_type=jnp.float32)
    for r in range(R):
        yr_ref[r] = jnp.dot(xb, wc_ref[r + 1],
                            preferred_element_type=jnp.float32)


def _comb_body(ry_ref, a_ref, b_ref, o_ref, ps_ref, psq_ref):
    o = ry_ref[...] + a_ref[0] + a_ref[1] + b_ref[...]
    o_ref[...] = o
    ps_ref[...] = jnp.sum(o, axis=0)[None, None]
    psq_ref[...] = jnp.sum(o * o, axis=0)[None, None]


def _bn_body(o_ref, ps_ref, psq_ref, g_ref, be_ref, out_ref):
    mu = jnp.sum(ps_ref[...], axis=(0, 1)) * (1.0 / N)
    ex2 = jnp.sum(psq_ref[...], axis=(0, 1)) * (1.0 / N)
    var = ex2 - mu * mu
    o = o_ref[...]
    h = (o - mu) * lax.rsqrt(var + EPS) * g_ref[...] + be_ref[...]
    sp = jnp.maximum(h, 0.0) + jnp.log1p(jnp.exp(-jnp.abs(h)))
    out_ref[...] = h * jnp.tanh(sp)


@jax.jit
def kernel(x, edge_index, edge_type, W, root, bias, gamma, beta):
    src = edge_index[0].astype(jnp.int32)
    dst = edge_index[1].astype(jnp.int32)
    typ = edge_type.astype(jnp.int32)
    wcat = jnp.concatenate([root[None], W], axis=0)

    root_y, yrel = pl.pallas_call(
        _mm_body,
        grid=(NB,),
        in_specs=[
            pl.BlockSpec((BN, D), lambda i: (i, 0)),
            pl.BlockSpec((R + 1, D, D), lambda i: (0, 0, 0)),
        ],
        out_specs=[
            pl.BlockSpec((BN, D), lambda i: (i, 0)),
            pl.BlockSpec((R, BN, D), lambda i: (0, i, 0)),
        ],
        out_shape=[
            jax.ShapeDtypeStruct((N, D), jnp.float32),
            jax.ShapeDtypeStruct((R, N, D), jnp.float32),
        ],
    )(x, wcat)

    yflat = yrel.reshape(R * N, D)
    acc = _sc_kernel(src, dst, typ, yflat)

    o, ps, psq = pl.pallas_call(
        _comb_body,
        grid=(NB,),
        in_specs=[
            pl.BlockSpec((BN, D), lambda i: (i, 0)),
            pl.BlockSpec((NC, BN, D), lambda i: (0, i, 0)),
            pl.BlockSpec((1, D), lambda i: (0, 0)),
        ],
        out_specs=[
            pl.BlockSpec((BN, D), lambda i: (i, 0)),
            pl.BlockSpec((1, 1, D), lambda i: (i, 0, 0)),
            pl.BlockSpec((1, 1, D), lambda i: (i, 0, 0)),
        ],
        out_shape=[
            jax.ShapeDtypeStruct((N, D), jnp.float32),
            jax.ShapeDtypeStruct((NB, 1, D), jnp.float32),
            jax.ShapeDtypeStruct((NB, 1, D), jnp.float32),
        ],
    )(root_y, acc, bias.reshape(1, D))

    out = pl.pallas_call(
        _bn_body,
        grid=(NB,),
        in_specs=[
            pl.BlockSpec((BN, D), lambda i: (i, 0)),
            pl.BlockSpec((NB, 1, D), lambda i: (0, 0, 0)),
            pl.BlockSpec((NB, 1, D), lambda i: (0, 0, 0)),
            pl.BlockSpec((1, D), lambda i: (0, 0)),
            pl.BlockSpec((1, D), lambda i: (0, 0)),
        ],
        out_specs=pl.BlockSpec((BN, D), lambda i: (i, 0)),
        out_shape=jax.ShapeDtypeStruct((N, D), jnp.float32),
    )(o, ps, psq, gamma.reshape(1, D), beta.reshape(1, D))
    return out


# speedup vs baseline: 1.1165x; 1.1165x over previous
import functools

import jax
import jax.numpy as jnp
from jax import lax
from jax.experimental import pallas as pl
from jax.experimental.pallas import tpu as pltpu
from jax.experimental.pallas import tpu_sc as plsc

N, E, D, R = 10000, 320000, 128, 8
NR = N * R
EPS = 1e-5

NC, NS = 2, 16
NW = NC * NS
CB = 80
BC = 2000
EPW = E // NW
EPC = E // NS
NRT = NR // NS
NRT_PAD = 5008
ART = 624
BN = 1000
NB = N // BN


def _iota16():
    return lax.broadcasted_iota(jnp.int32, (16,), 0)


def _sc_body(src_hbm, dst_hbm, typ_hbm, yflat_hbm, acc_out,
             cnt_sh, acc_sh, rowbuf, rowbuf1,
             srcB, dstB, typB,
             gidxb, dstb, widxb, gidxb1, dstb1, widxb1,
             wbuf, onesb, cidxb, cbuf, sem, sem1, sem2, sem3):
    c = lax.axis_index("c")
    s = lax.axis_index("s")
    wid = c * NS + s
    i16 = _iota16()
    z16 = jnp.zeros((16,), jnp.float32)
    o16 = jnp.ones((16,), jnp.float32)
    zi16 = jnp.zeros((16,), jnp.int32)

    def zero_row(e, _):
        for k in range(8):
            rowbuf[e, pl.ds(k * 16, 16)] = z16
        return 0
    lax.fori_loop(0, CB, zero_row, 0)

    def zero_cbuf(i, _):
        cbuf[pl.ds(i * 16, 16)] = z16
        return 0
    lax.fori_loop(0, NRT_PAD // 16, zero_cbuf, 0)

    for j in range(CB // 16):
        onesb[pl.ds(j * 16, 16)] = o16

    pltpu.sync_copy(cbuf.at[pl.ds(0, NRT)], cnt_sh.at[pl.ds(s * NRT, NRT)])
    for j in range(ART // CB):
        pltpu.sync_copy(rowbuf, acc_sh.at[pl.ds(s * ART + j * CB, CB)])
    rem = ART - (ART // CB) * CB
    if rem:
        pltpu.sync_copy(rowbuf.at[pl.ds(0, rem)],
                        acc_sh.at[pl.ds(s * ART + (ART // CB) * CB, rem)])

    @pl.when(s == NS - 1)
    def _zero_tail():
        pltpu.sync_copy(rowbuf.at[pl.ds(0, N - NS * ART)],
                        acc_sh.at[pl.ds(NS * ART, N - NS * ART)])
    plsc.subcore_barrier()

    cbase = s * EPC

    def cidx_chunk(t, ib):
        for j in range(CB // 16):
            off = t * CB + j * 16
            d16 = dstB[pl.ds(off, 16)]
            t16 = typB[pl.ds(off, 16)]
            ib[pl.ds(j * 16, 16)] = d16 * R + t16

    def count_block(b, _):
        pltpu.sync_copy(dst_hbm.at[pl.ds(cbase + b * BC, BC)], dstB)
        pltpu.sync_copy(typ_hbm.at[pl.ds(cbase + b * BC, BC)], typB)

        def count_pair(i, _):
            cidx_chunk(2 * i, cidxb)
            d0 = pltpu.async_copy(onesb, cnt_sh.at[cidxb], sem2, add=True)
            cidx_chunk(2 * i + 1, gidxb)
            d1 = pltpu.async_copy(onesb, cnt_sh.at[gidxb], sem3, add=True)
            d0.wait()
            d1.wait()
            return 0
        lax.fori_loop(0, BC // CB // 2, count_pair, 0)
        cidx_chunk(BC // CB - 1, cidxb)
        pltpu.sync_copy(onesb, cnt_sh.at[cidxb], add=True)
        return 0
    lax.fori_loop(0, EPC // BC, count_block, 0)
    plsc.subcore_barrier()

    pltpu.sync_copy(cnt_sh.at[pl.ds(s * NRT, NRT)], cbuf.at[pl.ds(0, NRT)])

    def inv_chunk(i, _):
        cv = cbuf[pl.ds(i * 16, 16)]
        cbuf[pl.ds(i * 16, 16)] = 1.0 / jnp.maximum(cv, 1.0)
        return 0
    lax.fori_loop(0, NRT_PAD // 16, inv_chunk, 0)
    pltpu.sync_copy(cbuf.at[pl.ds(0, NRT)], cnt_sh.at[pl.ds(s * NRT, NRT)])
    plsc.subcore_barrier()

    base = wid * EPW

    def idx_chunk(t, gb, db, wb):
        def g(j, _):
            off = t * CB + j * 16
            s16 = srcB[pl.ds(off, 16)]
            d16 = dstB[pl.ds(off, 16)]
            t16 = typB[pl.ds(off, 16)]
            gb[pl.ds(j * 16, 16)] = t16 * N + s16
            db[pl.ds(j * 16, 16)] = d16
            wb[pl.ds(j * 16, 16)] = d16 * R + t16
            return 0
        lax.fori_loop(0, CB // 16, g, 0)

    def scale(rb, wi):
        pltpu.sync_copy(cnt_sh.at[wi], wbuf)

        def scale_group(j, _):
            wv = wbuf[pl.ds(j * 16, 16)]
            for e in range(16):
                w = wv[e]
                row = j * 16 + e
                for k in range(8):
                    rb[row, pl.ds(k * 16, 16)] = rb[row, pl.ds(k * 16, 16)] * w
            return 0
        lax.fori_loop(0, CB // 16, scale_group, 0)

    def g_wait(rb, gb):
        pltpu.make_async_copy(yflat_hbm.at[gb], rb, sem).wait()

    def g_wait1(rb, gb):
        pltpu.make_async_copy(yflat_hbm.at[gb], rb, sem1).wait()

    CPB = BC // CB

    def block_body(b, _):
        pltpu.sync_copy(src_hbm.at[pl.ds(base + b * BC, BC)], srcB)
        pltpu.sync_copy(dst_hbm.at[pl.ds(base + b * BC, BC)], dstB)
        pltpu.sync_copy(typ_hbm.at[pl.ds(base + b * BC, BC)], typB)

        idx_chunk(0, gidxb, dstb, widxb)
        pltpu.async_copy(yflat_hbm.at[gidxb], rowbuf, sem)

        def pipe_body(i, _):
            t = 2 * i
            idx_chunk(t + 1, gidxb1, dstb1, widxb1)
            pltpu.async_copy(yflat_hbm.at[gidxb1], rowbuf1, sem1)
            g_wait(rowbuf, gidxb)
            scale(rowbuf, widxb)
            pltpu.sync_copy(rowbuf, acc_sh.at[dstb], add=True)
            idx_chunk(t + 2, gidxb, dstb, widxb)
            pltpu.async_copy(yflat_hbm.at[gidxb], rowbuf, sem)
            g_wait1(rowbuf1, gidxb1)
            scale(rowbuf1, widxb1)
            pltpu.sync_copy(rowbuf1, acc_sh.at[dstb1], add=True)
            return 0
        lax.fori_loop(0, (CPB - 1) // 2, pipe_body, 0)
        g_wait(rowbuf, gidxb)
        scale(rowbuf, widxb)
        pltpu.sync_copy(rowbuf, acc_sh.at[dstb], add=True)
        return 0
    lax.fori_loop(0, EPW // BC, block_body, 0)
    plsc.subcore_barrier()

    pltpu.sync_copy(acc_sh.at[pl.ds(s * ART, ART)],
                    acc_out.at[c, pl.ds(s * ART, ART)])

    @pl.when(s == NS - 1)
    def _export_tail():
        pltpu.sync_copy(acc_sh.at[pl.ds(NS * ART, N - NS * ART)],
                        acc_out.at[c, pl.ds(NS * ART, N - NS * ART)])


_sc_kernel = functools.partial(
    pl.kernel,
    out_type=jax.ShapeDtypeStruct((NC, N, D), jnp.float32),
    mesh=plsc.VectorSubcoreMesh(core_axis_name="c", subcore_axis_name="s",
                                num_cores=NC, num_subcores=NS),
    scratch_types=[
        pltpu.VMEM_SHARED((NR,), jnp.float32),
        pltpu.VMEM_SHARED((N, D), jnp.float32),
        pltpu.VMEM((CB, D), jnp.float32),
        pltpu.VMEM((CB, D), jnp.float32),
        pltpu.VMEM((BC,), jnp.int32),
        pltpu.VMEM((BC,), jnp.int32),
        pltpu.VMEM((BC,), jnp.int32),
        pltpu.VMEM((CB,), jnp.int32),
        pltpu.VMEM((CB,), jnp.int32),
        pltpu.VMEM((CB,), jnp.int32),
        pltpu.VMEM((CB,), jnp.int32),
        pltpu.VMEM((CB,), jnp.int32),
        pltpu.VMEM((CB,), jnp.int32),
        pltpu.VMEM((CB,), jnp.float32),
        pltpu.VMEM((CB,), jnp.float32),
        pltpu.VMEM((CB,), jnp.int32),
        pltpu.VMEM((NRT_PAD,), jnp.float32),
        pltpu.SemaphoreType.DMA,
        pltpu.SemaphoreType.DMA,
        pltpu.SemaphoreType.DMA,
        pltpu.SemaphoreType.DMA,
    ],
)(_sc_body)


def _mm_body(x_ref, w_ref, yr_ref):
    xb = x_ref[...]
    for r in range(R):
        yr_ref[r] = jnp.dot(xb, w_ref[r], preferred_element_type=jnp.float32)


def _fin_body(x_ref, root_ref, a_ref, b_ref, g_ref, be_ref, out_ref):
    o = jnp.dot(x_ref[...], root_ref[...], preferred_element_type=jnp.float32)
    o = o + a_ref[0] + a_ref[1] + b_ref[...]
    mu = jnp.mean(o, axis=0)
    var = jnp.mean(o * o, axis=0) - mu * mu
    h = (o - mu) * lax.rsqrt(var + EPS) * g_ref[...] + be_ref[...]
    sp = jnp.maximum(h, 0.0) + jnp.log1p(jnp.exp(-jnp.abs(h)))
    out_ref[...] = h * jnp.tanh(sp)


@jax.jit
def kernel(x, edge_index, edge_type, W, root, bias, gamma, beta):
    src = edge_index[0].astype(jnp.int32)
    dst = edge_index[1].astype(jnp.int32)
    typ = edge_type.astype(jnp.int32)

    yrel = pl.pallas_call(
        _mm_body,
        grid=(NB,),
        in_specs=[
            pl.BlockSpec((BN, D), lambda i: (i, 0)),
            pl.BlockSpec((R, D, D), lambda i: (0, 0, 0)),
        ],
        out_specs=pl.BlockSpec((R, BN, D), lambda i: (0, i, 0)),
        out_shape=jax.ShapeDtypeStruct((R, N, D), jnp.float32),
    )(x, W)

    yflat = yrel.reshape(R * N, D)
    acc = _sc_kernel(src, dst, typ, yflat)

    out = pl.pallas_call(
        _fin_body,
        out_shape=jax.ShapeDtypeStruct((N, D), jnp.float32),
    )(x, root, acc, bias.reshape(1, D), gamma.reshape(1, D),
      beta.reshape(1, D))
    return out


# speedup vs baseline: 1.1305x; 1.0125x over previous
import functools

import jax
import jax.numpy as jnp
from jax import lax
from jax.experimental import pallas as pl
from jax.experimental.pallas import tpu as pltpu
from jax.experimental.pallas import tpu_sc as plsc

N, E, D, R = 10000, 320000, 128, 8
NR = N * R
EPS = 1e-5

NC, NS = 2, 16
NW = NC * NS
CB = 80
BC = 2000
EPW = E // NW
EPC = E // NS
NRT = NR // NS
NRT_PAD = 5008
ART = 624
BN = 1000
NB = N // BN


def _iota16():
    return lax.broadcasted_iota(jnp.int32, (16,), 0)


def _sc_body(src_hbm, dst_hbm, typ_hbm, yflat_hbm, acc_out,
             cnt_sh, acc_sh, rowbuf, rowbuf1, rowbuf2,
             srcB, dstB, typB,
             gidxb, dstb, widxb, gidxb1, dstb1, widxb1,
             gidxb2, dstb2, widxb2,
             wbuf, onesb, cidxb, cbuf, sem, sem1, sem2, sem3, sem4):
    c = lax.axis_index("c")
    s = lax.axis_index("s")
    wid = c * NS + s
    i16 = _iota16()
    z16 = jnp.zeros((16,), jnp.float32)
    o16 = jnp.ones((16,), jnp.float32)
    zi16 = jnp.zeros((16,), jnp.int32)

    def zero_row(e, _):
        for k in range(8):
            rowbuf[e, pl.ds(k * 16, 16)] = z16
        return 0
    lax.fori_loop(0, CB, zero_row, 0)

    def zero_cbuf(i, _):
        cbuf[pl.ds(i * 16, 16)] = z16
        return 0
    lax.fori_loop(0, NRT_PAD // 16, zero_cbuf, 0)

    for j in range(CB // 16):
        onesb[pl.ds(j * 16, 16)] = o16

    pltpu.sync_copy(cbuf.at[pl.ds(0, NRT)], cnt_sh.at[pl.ds(s * NRT, NRT)])
    for j in range(ART // CB):
        pltpu.sync_copy(rowbuf, acc_sh.at[pl.ds(s * ART + j * CB, CB)])
    rem = ART - (ART // CB) * CB
    if rem:
        pltpu.sync_copy(rowbuf.at[pl.ds(0, rem)],
                        acc_sh.at[pl.ds(s * ART + (ART // CB) * CB, rem)])

    @pl.when(s == NS - 1)
    def _zero_tail():
        pltpu.sync_copy(rowbuf.at[pl.ds(0, N - NS * ART)],
                        acc_sh.at[pl.ds(NS * ART, N - NS * ART)])
    plsc.subcore_barrier()

    cbase = s * EPC

    def cidx_chunk(t, ib):
        for j in range(CB // 16):
            off = t * CB + j * 16
            d16 = dstB[pl.ds(off, 16)]
            t16 = typB[pl.ds(off, 16)]
            ib[pl.ds(j * 16, 16)] = d16 * R + t16

    def count_block(b, _):
        pltpu.sync_copy(dst_hbm.at[pl.ds(cbase + b * BC, BC)], dstB)
        pltpu.sync_copy(typ_hbm.at[pl.ds(cbase + b * BC, BC)], typB)

        def count_pair(i, _):
            cidx_chunk(2 * i, cidxb)
            d0 = pltpu.async_copy(onesb, cnt_sh.at[cidxb], sem2, add=True)
            cidx_chunk(2 * i + 1, gidxb)
            d1 = pltpu.async_copy(onesb, cnt_sh.at[gidxb], sem3, add=True)
            d0.wait()
            d1.wait()
            return 0
        lax.fori_loop(0, BC // CB // 2, count_pair, 0)
        cidx_chunk(BC // CB - 1, cidxb)
        pltpu.sync_copy(onesb, cnt_sh.at[cidxb], add=True)
        return 0
    lax.fori_loop(0, EPC // BC, count_block, 0)
    plsc.subcore_barrier()

    pltpu.sync_copy(cnt_sh.at[pl.ds(s * NRT, NRT)], cbuf.at[pl.ds(0, NRT)])

    def inv_chunk(i, _):
        cv = cbuf[pl.ds(i * 16, 16)]
        cbuf[pl.ds(i * 16, 16)] = 1.0 / jnp.maximum(cv, 1.0)
        return 0
    lax.fori_loop(0, NRT_PAD // 16, inv_chunk, 0)
    pltpu.sync_copy(cbuf.at[pl.ds(0, NRT)], cnt_sh.at[pl.ds(s * NRT, NRT)])
    plsc.subcore_barrier()

    base = wid * EPW

    def idx_chunk(t, gb, db, wb):
        def g(j, _):
            off = t * CB + j * 16
            s16 = srcB[pl.ds(off, 16)]
            d16 = dstB[pl.ds(off, 16)]
            t16 = typB[pl.ds(off, 16)]
            gb[pl.ds(j * 16, 16)] = t16 * N + s16
            db[pl.ds(j * 16, 16)] = d16
            wb[pl.ds(j * 16, 16)] = d16 * R + t16
            return 0
        lax.fori_loop(0, CB // 16, g, 0)

    def scale(rb, wi):
        pltpu.sync_copy(cnt_sh.at[wi], wbuf)

        def scale_group(j, _):
            wv = wbuf[pl.ds(j * 16, 16)]
            for e in range(16):
                w = wv[e]
                row = j * 16 + e
                for k in range(8):
                    rb[row, pl.ds(k * 16, 16)] = rb[row, pl.ds(k * 16, 16)] * w
            return 0
        lax.fori_loop(0, CB // 16, scale_group, 0)

    def g_wait(rb, gb, sm):
        pltpu.make_async_copy(yflat_hbm.at[gb], rb, sm).wait()

    CPB = BC // CB
    NG = (CPB - 1) // 3

    def block_body(b, _):
        pltpu.sync_copy(src_hbm.at[pl.ds(base + b * BC, BC)], srcB)
        pltpu.sync_copy(dst_hbm.at[pl.ds(base + b * BC, BC)], dstB)
        pltpu.sync_copy(typ_hbm.at[pl.ds(base + b * BC, BC)], typB)

        idx_chunk(0, gidxb, dstb, widxb)
        pltpu.async_copy(yflat_hbm.at[gidxb], rowbuf, sem)
        idx_chunk(1, gidxb1, dstb1, widxb1)
        pltpu.async_copy(yflat_hbm.at[gidxb1], rowbuf1, sem1)
        idx_chunk(2, gidxb2, dstb2, widxb2)
        pltpu.async_copy(yflat_hbm.at[gidxb2], rowbuf2, sem4)

        def pipe_body(g, _):
            t = 3 * g
            g_wait(rowbuf, gidxb, sem)
            scale(rowbuf, widxb)
            pltpu.sync_copy(rowbuf, acc_sh.at[dstb], add=True)
            idx_chunk(t + 3, gidxb, dstb, widxb)
            pltpu.async_copy(yflat_hbm.at[gidxb], rowbuf, sem)

            g_wait(rowbuf1, gidxb1, sem1)
            scale(rowbuf1, widxb1)
            pltpu.sync_copy(rowbuf1, acc_sh.at[dstb1], add=True)

            @pl.when(g < NG - 1)
            def _issue1():
                idx_chunk(t + 4, gidxb1, dstb1, widxb1)
                pltpu.async_copy(yflat_hbm.at[gidxb1], rowbuf1, sem1)

            g_wait(rowbuf2, gidxb2, sem4)
            scale(rowbuf2, widxb2)
            pltpu.sync_copy(rowbuf2, acc_sh.at[dstb2], add=True)

            @pl.when(g < NG - 1)
            def _issue2():
                idx_chunk(t + 5, gidxb2, dstb2, widxb2)
                pltpu.async_copy(yflat_hbm.at[gidxb2], rowbuf2, sem4)
            return 0
        lax.fori_loop(0, NG, pipe_body, 0)
        g_wait(rowbuf, gidxb, sem)
        scale(rowbuf, widxb)
        pltpu.sync_copy(rowbuf, acc_sh.at[dstb], add=True)
        return 0
    lax.fori_loop(0, EPW // BC, block_body, 0)
    plsc.subcore_barrier()

    pltpu.sync_copy(acc_sh.at[pl.ds(s * ART, ART)],
                    acc_out.at[c, pl.ds(s * ART, ART)])

    @pl.when(s == NS - 1)
    def _export_tail():
        pltpu.sync_copy(acc_sh.at[pl.ds(NS * ART, N - NS * ART)],
                        acc_out.at[c, pl.ds(NS * ART, N - NS * ART)])


_sc_kernel = functools.partial(
    pl.kernel,
    out_type=jax.ShapeDtypeStruct((NC, N, D), jnp.float32),
    mesh=plsc.VectorSubcoreMesh(core_axis_name="c", subcore_axis_name="s",
                                num_cores=NC, num_subcores=NS),
    scratch_types=[
        pltpu.VMEM_SHARED((NR,), jnp.float32),
        pltpu.VMEM_SHARED((N, D), jnp.float32),
        pltpu.VMEM((CB, D), jnp.float32),
        pltpu.VMEM((CB, D), jnp.float32),
        pltpu.VMEM((CB, D), jnp.float32),
        pltpu.VMEM((BC,), jnp.int32),
        pltpu.VMEM((BC,), jnp.int32),
        pltpu.VMEM((BC,), jnp.int32),
        pltpu.VMEM((CB,), jnp.int32),
        pltpu.VMEM((CB,), jnp.int32),
        pltpu.VMEM((CB,), jnp.int32),
        pltpu.VMEM((CB,), jnp.int32),
        pltpu.VMEM((CB,), jnp.int32),
        pltpu.VMEM((CB,), jnp.int32),
        pltpu.VMEM((CB,), jnp.int32),
        pltpu.VMEM((CB,), jnp.int32),
        pltpu.VMEM((CB,), jnp.int32),
        pltpu.VMEM((CB,), jnp.float32),
        pltpu.VMEM((CB,), jnp.float32),
        pltpu.VMEM((CB,), jnp.int32),
        pltpu.VMEM((NRT_PAD,), jnp.float32),
        pltpu.SemaphoreType.DMA,
        pltpu.SemaphoreType.DMA,
        pltpu.SemaphoreType.DMA,
        pltpu.SemaphoreType.DMA,
        pltpu.SemaphoreType.DMA,
    ],
)(_sc_body)


def _mm_body(x_ref, w_ref, yr_ref):
    xb = x_ref[...]
    for r in range(R):
        yr_ref[r] = jnp.dot(xb, w_ref[r], preferred_element_type=jnp.float32)


def _fin_body(x_ref, root_ref, a_ref, b_ref, g_ref, be_ref, out_ref):
    o = jnp.dot(x_ref[...], root_ref[...], preferred_element_type=jnp.float32)
    o = o + a_ref[0] + a_ref[1] + b_ref[...]
    mu = jnp.mean(o, axis=0)
    var = jnp.mean(o * o, axis=0) - mu * mu
    h = (o - mu) * lax.rsqrt(var + EPS) * g_ref[...] + be_ref[...]
    sp = jnp.maximum(h, 0.0) + jnp.log1p(jnp.exp(-jnp.abs(h)))
    out_ref[...] = h * jnp.tanh(sp)


@jax.jit
def kernel(x, edge_index, edge_type, W, root, bias, gamma, beta):
    src = edge_index[0].astype(jnp.int32)
    dst = edge_index[1].astype(jnp.int32)
    typ = edge_type.astype(jnp.int32)

    yrel = pl.pallas_call(
        _mm_body,
        grid=(NB,),
        in_specs=[
            pl.BlockSpec((BN, D), lambda i: (i, 0)),
            pl.BlockSpec((R, D, D), lambda i: (0, 0, 0)),
        ],
        out_specs=pl.BlockSpec((R, BN, D), lambda i: (0, i, 0)),
        out_shape=jax.ShapeDtypeStruct((R, N, D), jnp.float32),
    )(x, W)

    yflat = yrel.reshape(R * N, D)
    acc = _sc_kernel(src, dst, typ, yflat)

    out = pl.pallas_call(
        _fin_body,
        out_shape=jax.ShapeDtypeStruct((N, D), jnp.float32),
    )(x, root, acc, bias.reshape(1, D), gamma.reshape(1, D),
      beta.reshape(1, D))
    return out


# speedup vs baseline: 1.2116x; 1.0718x over previous
import functools

import jax
import jax.numpy as jnp
from jax import lax
from jax.experimental import pallas as pl
from jax.experimental.pallas import tpu as pltpu
from jax.experimental.pallas import tpu_sc as plsc

N, E, D, R = 10000, 320000, 128, 8
NR = N * R
EPS = 1e-5

NC, NS = 2, 16
NW = NC * NS
CB = 80
BC = 2000
EPW = E // NW
EPC = E // NS
NRT = NR // NS
NRT_PAD = 5008
ART = 624
BN = 1000
NB = N // BN


def _iota16():
    return lax.broadcasted_iota(jnp.int32, (16,), 0)


def _sc_body(src_hbm, dst_hbm, typ_hbm, yflat_hbm, acc_out,
             cnt_sh, acc_sh, rowbuf, rowbuf1, rowbuf2,
             srcB, dstB, typB,
             gidxb, dstb, widxb, gidxb1, dstb1, widxb1,
             gidxb2, dstb2, widxb2,
             wbuf, onesb, cidxb, cbuf, sem, sem1, sem2, sem3, sem4,
             sem5, sem6, sem7):
    c = lax.axis_index("c")
    s = lax.axis_index("s")
    wid = c * NS + s
    i16 = _iota16()
    z16 = jnp.zeros((16,), jnp.float32)
    o16 = jnp.ones((16,), jnp.float32)
    zi16 = jnp.zeros((16,), jnp.int32)

    def zero_row(e, _):
        for k in range(8):
            rowbuf[e, pl.ds(k * 16, 16)] = z16
        return 0
    lax.fori_loop(0, CB, zero_row, 0)

    def zero_cbuf(i, _):
        cbuf[pl.ds(i * 16, 16)] = z16
        return 0
    lax.fori_loop(0, NRT_PAD // 16, zero_cbuf, 0)

    for j in range(CB // 16):
        onesb[pl.ds(j * 16, 16)] = o16

    pltpu.sync_copy(cbuf.at[pl.ds(0, NRT)], cnt_sh.at[pl.ds(s * NRT, NRT)])
    for j in range(ART // CB):
        pltpu.sync_copy(rowbuf, acc_sh.at[pl.ds(s * ART + j * CB, CB)])
    rem = ART - (ART // CB) * CB
    if rem:
        pltpu.sync_copy(rowbuf.at[pl.ds(0, rem)],
                        acc_sh.at[pl.ds(s * ART + (ART // CB) * CB, rem)])

    @pl.when(s == NS - 1)
    def _zero_tail():
        pltpu.sync_copy(rowbuf.at[pl.ds(0, N - NS * ART)],
                        acc_sh.at[pl.ds(NS * ART, N - NS * ART)])
    plsc.subcore_barrier()

    cbase = s * EPC

    def cidx_chunk(t, ib):
        for j in range(CB // 16):
            off = t * CB + j * 16
            d16 = dstB[pl.ds(off, 16)]
            t16 = typB[pl.ds(off, 16)]
            ib[pl.ds(j * 16, 16)] = d16 * R + t16

    def count_block(b, _):
        pltpu.sync_copy(dst_hbm.at[pl.ds(cbase + b * BC, BC)], dstB)
        pltpu.sync_copy(typ_hbm.at[pl.ds(cbase + b * BC, BC)], typB)

        def count_pair(i, _):
            cidx_chunk(2 * i, cidxb)
            d0 = pltpu.async_copy(onesb, cnt_sh.at[cidxb], sem2, add=True)
            cidx_chunk(2 * i + 1, gidxb)
            d1 = pltpu.async_copy(onesb, cnt_sh.at[gidxb], sem3, add=True)
            d0.wait()
            d1.wait()
            return 0
        lax.fori_loop(0, BC // CB // 2, count_pair, 0)
        cidx_chunk(BC // CB - 1, cidxb)
        pltpu.sync_copy(onesb, cnt_sh.at[cidxb], add=True)
        return 0
    lax.fori_loop(0, EPC // BC, count_block, 0)
    plsc.subcore_barrier()

    pltpu.sync_copy(cnt_sh.at[pl.ds(s * NRT, NRT)], cbuf.at[pl.ds(0, NRT)])

    def inv_chunk(i, _):
        cv = cbuf[pl.ds(i * 16, 16)]
        cbuf[pl.ds(i * 16, 16)] = 1.0 / jnp.maximum(cv, 1.0)
        return 0
    lax.fori_loop(0, NRT_PAD // 16, inv_chunk, 0)
    pltpu.sync_copy(cbuf.at[pl.ds(0, NRT)], cnt_sh.at[pl.ds(s * NRT, NRT)])
    plsc.subcore_barrier()

    base = wid * EPW

    def idx_chunk(t, gb, db, wb):
        def g(j, _):
            off = t * CB + j * 16
            s16 = srcB[pl.ds(off, 16)]
            d16 = dstB[pl.ds(off, 16)]
            t16 = typB[pl.ds(off, 16)]
            gb[pl.ds(j * 16, 16)] = t16 * N + s16
            db[pl.ds(j * 16, 16)] = d16
            wb[pl.ds(j * 16, 16)] = d16 * R + t16
            return 0
        lax.fori_loop(0, CB // 16, g, 0)

    def scale(rb, wi):
        pltpu.sync_copy(cnt_sh.at[wi], wbuf)

        def scale_group(j, _):
            wv = wbuf[pl.ds(j * 16, 16)]
            for e in range(16):
                w = wv[e]
                row = j * 16 + e
                for k in range(8):
                    rb[row, pl.ds(k * 16, 16)] = rb[row, pl.ds(k * 16, 16)] * w
            return 0
        lax.fori_loop(0, CB // 16, scale_group, 0)

    def g_wait(rb, gb, sm):
        pltpu.make_async_copy(yflat_hbm.at[gb], rb, sm).wait()

    CPB = BC // CB
    NG = (CPB - 1) // 3

    def block_body(b, _):
        pltpu.sync_copy(src_hbm.at[pl.ds(base + b * BC, BC)], srcB)
        pltpu.sync_copy(dst_hbm.at[pl.ds(base + b * BC, BC)], dstB)
        pltpu.sync_copy(typ_hbm.at[pl.ds(base + b * BC, BC)], typB)

        idx_chunk(0, gidxb, dstb, widxb)
        pltpu.async_copy(yflat_hbm.at[gidxb], rowbuf, sem)
        idx_chunk(1, gidxb1, dstb1, widxb1)
        pltpu.async_copy(yflat_hbm.at[gidxb1], rowbuf1, sem1)
        idx_chunk(2, gidxb2, dstb2, widxb2)
        pltpu.async_copy(yflat_hbm.at[gidxb2], rowbuf2, sem4)

        def pipe_body(g, _):
            t = 3 * g
            g_wait(rowbuf, gidxb, sem)
            scale(rowbuf, widxb)
            sc0 = pltpu.async_copy(rowbuf, acc_sh.at[dstb], sem5, add=True)

            g_wait(rowbuf1, gidxb1, sem1)
            scale(rowbuf1, widxb1)
            sc1 = pltpu.async_copy(rowbuf1, acc_sh.at[dstb1], sem6, add=True)

            sc0.wait()
            idx_chunk(t + 3, gidxb, dstb, widxb)
            pltpu.async_copy(yflat_hbm.at[gidxb], rowbuf, sem)

            g_wait(rowbuf2, gidxb2, sem4)
            scale(rowbuf2, widxb2)
            sc2 = pltpu.async_copy(rowbuf2, acc_sh.at[dstb2], sem7, add=True)

            sc1.wait()

            @pl.when(g < NG - 1)
            def _issue1():
                idx_chunk(t + 4, gidxb1, dstb1, widxb1)
                pltpu.async_copy(yflat_hbm.at[gidxb1], rowbuf1, sem1)

            sc2.wait()

            @pl.when(g < NG - 1)
            def _issue2():
                idx_chunk(t + 5, gidxb2, dstb2, widxb2)
                pltpu.async_copy(yflat_hbm.at[gidxb2], rowbuf2, sem4)
            return 0
        lax.fori_loop(0, NG, pipe_body, 0)
        g_wait(rowbuf, gidxb, sem)
        scale(rowbuf, widxb)
        pltpu.sync_copy(rowbuf, acc_sh.at[dstb], add=True)
        return 0
    lax.fori_loop(0, EPW // BC, block_body, 0)
    plsc.subcore_barrier()

    pltpu.sync_copy(acc_sh.at[pl.ds(s * ART, ART)],
                    acc_out.at[c, pl.ds(s * ART, ART)])

    @pl.when(s == NS - 1)
    def _export_tail():
        pltpu.sync_copy(acc_sh.at[pl.ds(NS * ART, N - NS * ART)],
                        acc_out.at[c, pl.ds(NS * ART, N - NS * ART)])


_sc_kernel = functools.partial(
    pl.kernel,
    out_type=jax.ShapeDtypeStruct((NC, N, D), jnp.float32),
    mesh=plsc.VectorSubcoreMesh(core_axis_name="c", subcore_axis_name="s",
                                num_cores=NC, num_subcores=NS),
    scratch_types=[
        pltpu.VMEM_SHARED((NR,), jnp.float32),
        pltpu.VMEM_SHARED((N, D), jnp.float32),
        pltpu.VMEM((CB, D), jnp.float32),
        pltpu.VMEM((CB, D), jnp.float32),
        pltpu.VMEM((CB, D), jnp.float32),
        pltpu.VMEM((BC,), jnp.int32),
        pltpu.VMEM((BC,), jnp.int32),
        pltpu.VMEM((BC,), jnp.int32),
        pltpu.VMEM((CB,), jnp.int32),
        pltpu.VMEM((CB,), jnp.int32),
        pltpu.VMEM((CB,), jnp.int32),
        pltpu.VMEM((CB,), jnp.int32),
        pltpu.VMEM((CB,), jnp.int32),
        pltpu.VMEM((CB,), jnp.int32),
        pltpu.VMEM((CB,), jnp.int32),
        pltpu.VMEM((CB,), jnp.int32),
        pltpu.VMEM((CB,), jnp.int32),
        pltpu.VMEM((CB,), jnp.float32),
        pltpu.VMEM((CB,), jnp.float32),
        pltpu.VMEM((CB,), jnp.int32),
        pltpu.VMEM((NRT_PAD,), jnp.float32),
        pltpu.SemaphoreType.DMA,
        pltpu.SemaphoreType.DMA,
        pltpu.SemaphoreType.DMA,
        pltpu.SemaphoreType.DMA,
        pltpu.SemaphoreType.DMA,
        pltpu.SemaphoreType.DMA,
        pltpu.SemaphoreType.DMA,
        pltpu.SemaphoreType.DMA,
    ],
)(_sc_body)


def _mm_body(x_ref, w_ref, yr_ref):
    xb = x_ref[...]
    for r in range(R):
        yr_ref[r] = jnp.dot(xb, w_ref[r], preferred_element_type=jnp.float32)


def _fin_body(x_ref, root_ref, a_ref, b_ref, g_ref, be_ref, out_ref):
    o = jnp.dot(x_ref[...], root_ref[...], preferred_element_type=jnp.float32)
    o = o + a_ref[0] + a_ref[1] + b_ref[...]
    mu = jnp.mean(o, axis=0)
    var = jnp.mean(o * o, axis=0) - mu * mu
    h = (o - mu) * lax.rsqrt(var + EPS) * g_ref[...] + be_ref[...]
    sp = jnp.maximum(h, 0.0) + jnp.log1p(jnp.exp(-jnp.abs(h)))
    out_ref[...] = h * jnp.tanh(sp)


@jax.jit
def kernel(x, edge_index, edge_type, W, root, bias, gamma, beta):
    src = edge_index[0].astype(jnp.int32)
    dst = edge_index[1].astype(jnp.int32)
    typ = edge_type.astype(jnp.int32)

    yrel = pl.pallas_call(
        _mm_body,
        grid=(NB,),
        in_specs=[
            pl.BlockSpec((BN, D), lambda i: (i, 0)),
            pl.BlockSpec((R, D, D), lambda i: (0, 0, 0)),
        ],
        out_specs=pl.BlockSpec((R, BN, D), lambda i: (0, i, 0)),
        out_shape=jax.ShapeDtypeStruct((R, N, D), jnp.float32),
    )(x, W)

    yflat = yrel.reshape(R * N, D)
    acc = _sc_kernel(src, dst, typ, yflat)

    out = pl.pallas_call(
        _fin_body,
        out_shape=jax.ShapeDtypeStruct((N, D), jnp.float32),
    )(x, root, acc, bias.reshape(1, D), gamma.reshape(1, D),
      beta.reshape(1, D))
    return out


# speedup vs baseline: 1.2124x; 1.0006x over previous
import functools

import jax
import jax.numpy as jnp
from jax import lax
from jax.experimental import pallas as pl
from jax.experimental.pallas import tpu as pltpu
from jax.experimental.pallas import tpu_sc as plsc

N, E, D, R = 10000, 320000, 128, 8
NR = N * R
EPS = 1e-5

NC, NS = 2, 16
NW = NC * NS
CB = 80
BC = 2000
EPW = E // NW
EPC = E // NS
NRT = NR // NS
NRT_PAD = 5008
ART = 624
BN = 1000
NB = N // BN


def _iota16():
    return lax.broadcasted_iota(jnp.int32, (16,), 0)


def _sc_body(src_hbm, dst_hbm, typ_hbm, yflat_hbm, acc_out,
             cnt_sh, acc_sh, rowbuf, rowbuf1, rowbuf2,
             srcB, dstB, typB,
             gidxb, dstb, widxb, gidxb1, dstb1, widxb1,
             gidxb2, dstb2, widxb2,
             wbuf, onesb, cidxb, cbuf, sem, sem1, sem2, sem3, sem4,
             sem5, sem6, sem7):
    c = lax.axis_index("c")
    s = lax.axis_index("s")
    wid = c * NS + s
    i16 = _iota16()
    z16 = jnp.zeros((16,), jnp.float32)
    o16 = jnp.ones((16,), jnp.float32)
    zi16 = jnp.zeros((16,), jnp.int32)

    def zero_row(e, _):
        for k in range(8):
            rowbuf[e, pl.ds(k * 16, 16)] = z16
        return 0
    lax.fori_loop(0, CB, zero_row, 0)

    def zero_cbuf(i, _):
        cbuf[pl.ds(i * 16, 16)] = z16
        return 0
    lax.fori_loop(0, NRT_PAD // 16, zero_cbuf, 0)

    for j in range(CB // 16):
        onesb[pl.ds(j * 16, 16)] = o16

    pltpu.sync_copy(cbuf.at[pl.ds(0, NRT)], cnt_sh.at[pl.ds(s * NRT, NRT)])
    for j in range(ART // CB):
        pltpu.sync_copy(rowbuf, acc_sh.at[pl.ds(s * ART + j * CB, CB)])
    rem = ART - (ART // CB) * CB
    if rem:
        pltpu.sync_copy(rowbuf.at[pl.ds(0, rem)],
                        acc_sh.at[pl.ds(s * ART + (ART // CB) * CB, rem)])

    @pl.when(s == NS - 1)
    def _zero_tail():
        pltpu.sync_copy(rowbuf.at[pl.ds(0, N - NS * ART)],
                        acc_sh.at[pl.ds(NS * ART, N - NS * ART)])
    plsc.subcore_barrier()

    cbase = s * EPC

    def cidx_chunk(t, ib):
        for j in range(CB // 16):
            off = t * CB + j * 16
            d16 = dstB[pl.ds(off, 16)]
            t16 = typB[pl.ds(off, 16)]
            ib[pl.ds(j * 16, 16)] = d16 * R + t16

    def count_block(b, _):
        pltpu.sync_copy(dst_hbm.at[pl.ds(cbase + b * BC, BC)], dstB)
        pltpu.sync_copy(typ_hbm.at[pl.ds(cbase + b * BC, BC)], typB)

        def count_pair(i, _):
            cidx_chunk(2 * i, cidxb)
            d0 = pltpu.async_copy(onesb, cnt_sh.at[cidxb], sem2, add=True)
            cidx_chunk(2 * i + 1, gidxb)
            d1 = pltpu.async_copy(onesb, cnt_sh.at[gidxb], sem3, add=True)
            d0.wait()
            d1.wait()
            return 0
        lax.fori_loop(0, BC // CB // 2, count_pair, 0)
        cidx_chunk(BC // CB - 1, cidxb)
        pltpu.sync_copy(onesb, cnt_sh.at[cidxb], add=True)
        return 0
    lax.fori_loop(0, EPC // BC, count_block, 0)
    plsc.subcore_barrier()

    pltpu.sync_copy(cnt_sh.at[pl.ds(s * NRT, NRT)], cbuf.at[pl.ds(0, NRT)])

    def inv_chunk(i, _):
        cv = cbuf[pl.ds(i * 16, 16)]
        cbuf[pl.ds(i * 16, 16)] = 1.0 / jnp.maximum(cv, 1.0)
        return 0
    lax.fori_loop(0, NRT_PAD // 16, inv_chunk, 0)
    pltpu.sync_copy(cbuf.at[pl.ds(0, NRT)], cnt_sh.at[pl.ds(s * NRT, NRT)])
    plsc.subcore_barrier()

    base = wid * EPW

    def idx_chunk(t, gb, db, wb):
        def g(j, _):
            off = t * CB + j * 16
            s16 = srcB[pl.ds(off, 16)]
            d16 = dstB[pl.ds(off, 16)]
            t16 = typB[pl.ds(off, 16)]
            gb[pl.ds(j * 16, 16)] = t16 * N + s16
            db[pl.ds(j * 16, 16)] = d16
            wb[pl.ds(j * 16, 16)] = d16 * R + t16
            return 0
        lax.fori_loop(0, CB // 16, g, 0)

    def scale(rb, wi):
        pltpu.sync_copy(cnt_sh.at[wi], wbuf)

        def scale_group(j, _):
            wv = wbuf[pl.ds(j * 16, 16)]
            for e in range(16):
                w = wv[e]
                row = j * 16 + e
                for k in range(8):
                    rb[row, pl.ds(k * 16, 16)] = rb[row, pl.ds(k * 16, 16)] * w
            return 0
        lax.fori_loop(0, CB // 16, scale_group, 0)

    def g_wait(rb, gb, sm):
        pltpu.make_async_copy(yflat_hbm.at[gb], rb, sm).wait()

    CPB = BC // CB
    NG = (CPB - 1) // 3

    def block_body(b, _):
        pltpu.sync_copy(src_hbm.at[pl.ds(base + b * BC, BC)], srcB)
        pltpu.sync_copy(dst_hbm.at[pl.ds(base + b * BC, BC)], dstB)
        pltpu.sync_copy(typ_hbm.at[pl.ds(base + b * BC, BC)], typB)

        idx_chunk(0, gidxb, dstb, widxb)
        pltpu.async_copy(yflat_hbm.at[gidxb], rowbuf, sem)
        idx_chunk(1, gidxb1, dstb1, widxb1)
        pltpu.async_copy(yflat_hbm.at[gidxb1], rowbuf1, sem1)

        def pipe_body(g, _):
            t = 3 * g

            @pl.when(g > 0)
            def _drain2():
                pltpu.make_async_copy(rowbuf2, acc_sh.at[dstb2], sem7).wait()
            idx_chunk(t + 2, gidxb2, dstb2, widxb2)
            pltpu.async_copy(yflat_hbm.at[gidxb2], rowbuf2, sem4)

            g_wait(rowbuf, gidxb, sem)
            scale(rowbuf, widxb)
            sc0 = pltpu.async_copy(rowbuf, acc_sh.at[dstb], sem5, add=True)

            g_wait(rowbuf1, gidxb1, sem1)
            scale(rowbuf1, widxb1)
            sc1 = pltpu.async_copy(rowbuf1, acc_sh.at[dstb1], sem6, add=True)

            sc0.wait()
            idx_chunk(t + 3, gidxb, dstb, widxb)
            pltpu.async_copy(yflat_hbm.at[gidxb], rowbuf, sem)

            g_wait(rowbuf2, gidxb2, sem4)
            scale(rowbuf2, widxb2)
            pltpu.async_copy(rowbuf2, acc_sh.at[dstb2], sem7, add=True)

            sc1.wait()

            @pl.when(g < NG - 1)
            def _issue1():
                idx_chunk(t + 4, gidxb1, dstb1, widxb1)
                pltpu.async_copy(yflat_hbm.at[gidxb1], rowbuf1, sem1)
            return 0
        lax.fori_loop(0, NG, pipe_body, 0)
        pltpu.make_async_copy(rowbuf2, acc_sh.at[dstb2], sem7).wait()
        g_wait(rowbuf, gidxb, sem)
        scale(rowbuf, widxb)
        pltpu.sync_copy(rowbuf, acc_sh.at[dstb], add=True)
        return 0
    lax.fori_loop(0, EPW // BC, block_body, 0)
    plsc.subcore_barrier()

    pltpu.sync_copy(acc_sh.at[pl.ds(s * ART, ART)],
                    acc_out.at[c, pl.ds(s * ART, ART)])

    @pl.when(s == NS - 1)
    def _export_tail():
        pltpu.sync_copy(acc_sh.at[pl.ds(NS * ART, N - NS * ART)],
                        acc_out.at[c, pl.ds(NS * ART, N - NS * ART)])


_sc_kernel = functools.partial(
    pl.kernel,
    out_type=jax.ShapeDtypeStruct((NC, N, D), jnp.float32),
    mesh=plsc.VectorSubcoreMesh(core_axis_name="c", subcore_axis_name="s",
                                num_cores=NC, num_subcores=NS),
    scratch_types=[
        pltpu.VMEM_SHARED((NR,), jnp.float32),
        pltpu.VMEM_SHARED((N, D), jnp.float32),
        pltpu.VMEM((CB, D), jnp.float32),
        pltpu.VMEM((CB, D), jnp.float32),
        pltpu.VMEM((CB, D), jnp.float32),
        pltpu.VMEM((BC,), jnp.int32),
        pltpu.VMEM((BC,), jnp.int32),
        pltpu.VMEM((BC,), jnp.int32),
        pltpu.VMEM((CB,), jnp.int32),
        pltpu.VMEM((CB,), jnp.int32),
        pltpu.VMEM((CB,), jnp.int32),
        pltpu.VMEM((CB,), jnp.int32),
        pltpu.VMEM((CB,), jnp.int32),
        pltpu.VMEM((CB,), jnp.int32),
        pltpu.VMEM((CB,), jnp.int32),
        pltpu.VMEM((CB,), jnp.int32),
        pltpu.VMEM((CB,), jnp.int32),
        pltpu.VMEM((CB,), jnp.float32),
        pltpu.VMEM((CB,), jnp.float32),
        pltpu.VMEM((CB,), jnp.int32),
        pltpu.VMEM((NRT_PAD,), jnp.float32),
        pltpu.SemaphoreType.DMA,
        pltpu.SemaphoreType.DMA,
        pltpu.SemaphoreType.DMA,
        pltpu.SemaphoreType.DMA,
        pltpu.SemaphoreType.DMA,
        pltpu.SemaphoreType.DMA,
        pltpu.SemaphoreType.DMA,
        pltpu.SemaphoreType.DMA,
    ],
)(_sc_body)


def _mm_body(x_ref, w_ref, yr_ref):
    xb = x_ref[...]
    for r in range(R):
        yr_ref[r] = jnp.dot(xb, w_ref[r], preferred_element_type=jnp.float32)


def _fin_body(x_ref, root_ref, a_ref, b_ref, g_ref, be_ref, out_ref):
    o = jnp.dot(x_ref[...], root_ref[...], preferred_element_type=jnp.float32)
    o = o + a_ref[0] + a_ref[1] + b_ref[...]
    mu = jnp.mean(o, axis=0)
    var = jnp.mean(o * o, axis=0) - mu * mu
    h = (o - mu) * lax.rsqrt(var + EPS) * g_ref[...] + be_ref[...]
    sp = jnp.maximum(h, 0.0) + jnp.log1p(jnp.exp(-jnp.abs(h)))
    out_ref[...] = h * jnp.tanh(sp)


@jax.jit
def kernel(x, edge_index, edge_type, W, root, bias, gamma, beta):
    src = edge_index[0].astype(jnp.int32)
    dst = edge_index[1].astype(jnp.int32)
    typ = edge_type.astype(jnp.int32)

    yrel = pl.pallas_call(
        _mm_body,
        grid=(NB,),
        in_specs=[
            pl.BlockSpec((BN, D), lambda i: (i, 0)),
            pl.BlockSpec((R, D, D), lambda i: (0, 0, 0)),
        ],
        out_specs=pl.BlockSpec((R, BN, D), lambda i: (0, i, 0)),
        out_shape=jax.ShapeDtypeStruct((R, N, D), jnp.float32),
    )(x, W)

    yflat = yrel.reshape(R * N, D)
    acc = _sc_kernel(src, dst, typ, yflat)

    out = pl.pallas_call(
        _fin_body,
        out_shape=jax.ShapeDtypeStruct((N, D), jnp.float32),
    )(x, root, acc, bias.reshape(1, D), gamma.reshape(1, D),
      beta.reshape(1, D))
    return out
